# Initial kernel scaffold; baseline (speedup 1.0000x reference)
import jax
import jax.numpy as jnp
from jax.experimental import pallas as pl


def kernel(pos, normals, batch, params):
    raise NotImplementedError("write your pallas kernel here")


# speedup vs baseline: 3.5284x; 3.5284x over previous
import jax
import jax.numpy as jnp
import numpy as np
from jax.experimental import pallas as pl
from jax.experimental.pallas import tpu as pltpu

N = 10000
B = 10
NPG = 1000
K = 20
EPS = 1e-5

_I32_MAX = np.int32(2**31 - 1)


def _cdot(a, w):
    f32 = jnp.float32
    kd = a.shape[1]
    acc = jnp.dot(a[:, 0:min(128, kd)], w[0:min(128, kd), :],
                  preferred_element_type=f32)
    for c in range(128, kd, 128):
        e = min(c + 128, kd)
        acc = acc + jnp.dot(a[:, c:e], w[c:e, :], preferred_element_type=f32)
    return acc


def _edge_layer_body(x_ref, sqc_ref, sqr_ref, m3_ref, s_ref,
                     w1_ref, b1_ref, g1_ref, bt1_ref,
                     w2_ref, b2_ref, g2_ref, bt2_ref,
                     wg_ref, bg_ref, wb_ref, bb_ref,
                     y_ref, pool_ref, p_ref):
    f32 = jnp.float32
    x = x_ref[0]
    feat = x.shape[1]
    s = s_ref[0, 0]

    gram = jax.lax.dot_general(x, x, (((1,), (1,)), ((), ())),
                               preferred_element_type=f32)
    d = (sqc_ref[0] + sqr_ref[0]) - 2.0 * gram
    col = jax.lax.broadcasted_iota(jnp.int32, d.shape, 1)

    m3 = m3_ref[0]
    w1 = w1_ref[...]
    b1 = b1_ref[...]
    g1 = g1_ref[...]
    bt1 = bt1_ref[...]
    w2 = w2_ref[...]
    b2 = b2_ref[...]
    g2 = g2_ref[...]
    bt2 = bt2_ref[...]

    acc = jnp.full((NPG, w2.shape[1]), -jnp.inf, f32)
    for _ in range(K):
        m = jnp.min(d, axis=1, keepdims=True)
        cmin = jnp.min(jnp.where(d == m, col, _I32_MAX), axis=1, keepdims=True)
        oh = col == cmin
        ohf = jnp.where(oh, 1.0, 0.0).astype(f32)
        p_ref[...] = jnp.dot(ohf, m3, preferred_element_type=f32)
        xj = (p_ref[:, 0:feat] + p_ref[:, feat:2 * feat]) + p_ref[:, 2 * feat:]
        cat = jnp.concatenate([x, xj - x], axis=1)
        z = _cdot(cat, w1) + b1
        h = jnp.maximum(g1 * z / s + bt1, 0.0)
        z = _cdot(h, w2) + b2
        h = jnp.maximum(g2 * z / s + bt2, 0.0)
        acc = jnp.maximum(acc, h)
        d = jnp.where(oh, jnp.inf, d)

    gamma = _cdot(acc, wg_ref[...]) + bg_ref[...]
    beta = _cdot(acc, wb_ref[...]) + bb_ref[...]
    y = gamma * acc + beta
    y_ref[0] = y
    pool_ref[0] = jnp.sum(y, axis=0, keepdims=True) / jnp.float32(NPG)


def _edge_layer(x, s, w1, b1, g1, bt1, w2, b2, g2, bt2, wg, bg, wb, bb):
    feat = x.shape[-1]
    hid = w2.shape[-1]
    xg = x.reshape(B, NPG, feat)

    sq = jnp.stack([jnp.sum(xg[b] * xg[b], axis=1) for b in range(B)])
    sqc = sq.reshape(B, NPG, 1)
    sqr = sq.reshape(B, 1, NPG)

    hi = x.astype(jnp.bfloat16).astype(jnp.float32)
    lo1 = x - hi
    mid = lo1.astype(jnp.bfloat16).astype(jnp.float32)
    lo2 = lo1 - mid
    m3 = jnp.concatenate([hi, mid, lo2], axis=1).reshape(B, NPG, 3 * feat)

    def fixed(a):
        return pl.BlockSpec(a.shape, lambda b: (0,) * a.ndim)

    consts = (s, w1, b1, g1, bt1, w2, b2, g2, bt2, wg, bg, wb, bb)
    y, pool = pl.pallas_call(
        _edge_layer_body,
        grid=(B,),
        in_specs=[pl.BlockSpec((1, NPG, feat), lambda b: (b, 0, 0)),
                  pl.BlockSpec((1, NPG, 1), lambda b: (b, 0, 0)),
                  pl.BlockSpec((1, 1, NPG), lambda b: (b, 0, 0)),
                  pl.BlockSpec((1, NPG, 3 * feat), lambda b: (b, 0, 0))]
                 + [fixed(a) for a in consts],
        out_specs=[
            pl.BlockSpec((1, NPG, hid), lambda b: (b, 0, 0)),
            pl.BlockSpec((1, 1, hid), lambda b: (b, 0, 0)),
        ],
        out_shape=[
            jax.ShapeDtypeStruct((B, NPG, hid), jnp.float32),
            jax.ShapeDtypeStruct((B, 1, hid), jnp.float32),
        ],
        scratch_shapes=[pltpu.VMEM((NPG, 3 * feat), jnp.float32)],
    )(xg, sqc, sqr, m3, *consts)
    return y.reshape(N, hid), pool.reshape(B, hid)


def _head_body(h_ref, s_ref, w1_ref, b1_ref, g1_ref, bt1_ref,
               w2_ref, b2_ref, g2_ref, bt2_ref, w3_ref, b3_ref, out_ref):
    s = s_ref[0, 0]
    h = _cdot(h_ref[...], w1_ref[...]) + b1_ref[...]
    h = jnp.maximum(g1_ref[...] * h / s + bt1_ref[...], 0.0)
    h = _cdot(h, w2_ref[...]) + b2_ref[...]
    h = jnp.maximum(g2_ref[...] * h / s + bt2_ref[...], 0.0)
    out_ref[...] = _cdot(h, w3_ref[...]) + b3_ref[...]


@jax.jit
def kernel(pos, normals, batch, params):
    del batch
    x = jnp.concatenate([pos, normals], axis=1)
    s = jnp.sqrt(1.0 + EPS).astype(jnp.float32).reshape(1, 1)

    pools = []
    for i in (1, 2, 3, 4):
        c1, c2 = params[f'conv{i}']
        fl = params[f'film{i}']
        x, p = _edge_layer(
            x, s, c1['W'], c1['b'][None, :], c1['g'][None, :],
            c1['beta'][None, :], c2['W'], c2['b'][None, :],
            c2['g'][None, :], c2['beta'][None, :],
            fl['Wg'], fl['bg'][None, :], fl['Wb'], fl['bb'][None, :])
        pools.append(p)

    hcat = jnp.concatenate(pools, axis=1)
    return pl.pallas_call(
        _head_body,
        out_shape=jax.ShapeDtypeStruct((B, 1), jnp.float32),
    )(hcat, s,
      params['lin1']['W'], params['lin1']['b'][None, :],
      params['bn1']['g'][None, :], params['bn1']['beta'][None, :],
      params['lin2']['W'], params['lin2']['b'][None, :],
      params['bn2']['g'][None, :], params['bn2']['beta'][None, :],
      params['lin3']['W'], params['lin3']['b'][None, :])


# speedup vs baseline: 3.5310x; 1.0007x over previous
import jax
import jax.numpy as jnp
import numpy as np
from jax.experimental import pallas as pl
from jax.experimental.pallas import tpu as pltpu

N = 10000
B = 10
NPG = 1000
K = 20
EPS = 1e-5

_I32_MAX = np.int32(2**31 - 1)


def _cdot(a, w):
    f32 = jnp.float32
    kd = a.shape[1]
    acc = jnp.dot(a[:, 0:min(128, kd)], w[0:min(128, kd), :],
                  preferred_element_type=f32)
    for c in range(128, kd, 128):
        e = min(c + 128, kd)
        acc = acc + jnp.dot(a[:, c:e], w[c:e, :], preferred_element_type=f32)
    return acc


def _edge_layer_body(x_ref, sqc_ref, sqr_ref, m3_ref, s_ref,
                     w1_ref, b1_ref, g1_ref, bt1_ref,
                     w2_ref, b2_ref, g2_ref, bt2_ref,
                     wg_ref, bg_ref, wb_ref, bb_ref,
                     y_ref, pool_ref, p_ref):
    f32 = jnp.float32
    x = x_ref[0]
    feat = x.shape[1]
    s = s_ref[0, 0]

    gram = jax.lax.dot_general(x, x, (((1,), (1,)), ((), ())),
                               preferred_element_type=f32)
    d = (sqc_ref[0] + sqr_ref[0]) - 2.0 * gram
    col = jax.lax.broadcasted_iota(jnp.int32, d.shape, 1)

    m3 = m3_ref[0]
    w1 = w1_ref[...]
    b1 = b1_ref[...]
    g1 = g1_ref[...]
    bt1 = bt1_ref[...]
    w2 = w2_ref[...]
    b2 = b2_ref[...]
    g2 = g2_ref[...]
    bt2 = bt2_ref[...]

    acc = jnp.full((NPG, w2.shape[1]), -jnp.inf, f32)
    for _ in range(K):
        m = jnp.min(d, axis=1, keepdims=True)
        cmin = jnp.min(jnp.where(d == m, col, _I32_MAX), axis=1, keepdims=True)
        oh = col == cmin
        ohf = jnp.where(oh, 1.0, 0.0).astype(f32)
        p_ref[...] = jnp.dot(ohf, m3, preferred_element_type=f32)
        xj = (p_ref[:, 0:feat] + p_ref[:, feat:2 * feat]) + p_ref[:, 2 * feat:]
        cat = jnp.concatenate([x, xj - x], axis=1)
        z = _cdot(cat, w1) + b1
        h = jnp.maximum(g1 * z / s + bt1, 0.0)
        z = _cdot(h, w2) + b2
        h = jnp.maximum(g2 * z / s + bt2, 0.0)
        acc = jnp.maximum(acc, h)
        d = jnp.where(oh, jnp.inf, d)

    gamma = _cdot(acc, wg_ref[...]) + bg_ref[...]
    beta = _cdot(acc, wb_ref[...]) + bb_ref[...]
    y = gamma * acc + beta
    y_ref[0] = y
    pool_ref[0] = jnp.sum(y, axis=0, keepdims=True) / jnp.float32(NPG)


def _edge_layer(x, s, w1, b1, g1, bt1, w2, b2, g2, bt2, wg, bg, wb, bb):
    feat = x.shape[-1]
    hid = w2.shape[-1]
    xg = x.reshape(B, NPG, feat)

    sq = jnp.stack([jnp.sum(xg[b] * xg[b], axis=1) for b in range(B)])
    sqc = sq.reshape(B, NPG, 1)
    sqr = sq.reshape(B, 1, NPG)

    hi = x.astype(jnp.bfloat16).astype(jnp.float32)
    lo1 = x - hi
    mid = lo1.astype(jnp.bfloat16).astype(jnp.float32)
    lo2 = lo1 - mid
    m3 = jnp.concatenate([hi, mid, lo2], axis=1).reshape(B, NPG, 3 * feat)

    def fixed(a):
        return pl.BlockSpec(a.shape, lambda b: (0,) * a.ndim)

    consts = (s, w1, b1, g1, bt1, w2, b2, g2, bt2, wg, bg, wb, bb)
    y, pool = pl.pallas_call(
        _edge_layer_body,
        grid=(B,),
        in_specs=[pl.BlockSpec((1, NPG, feat), lambda b: (b, 0, 0)),
                  pl.BlockSpec((1, NPG, 1), lambda b: (b, 0, 0)),
                  pl.BlockSpec((1, 1, NPG), lambda b: (b, 0, 0)),
                  pl.BlockSpec((1, NPG, 3 * feat), lambda b: (b, 0, 0))]
                 + [fixed(a) for a in consts],
        out_specs=[
            pl.BlockSpec((1, NPG, hid), lambda b: (b, 0, 0)),
            pl.BlockSpec((1, 1, hid), lambda b: (b, 0, 0)),
        ],
        out_shape=[
            jax.ShapeDtypeStruct((B, NPG, hid), jnp.float32),
            jax.ShapeDtypeStruct((B, 1, hid), jnp.float32),
        ],
        scratch_shapes=[pltpu.VMEM((NPG, 3 * feat), jnp.float32)],
        compiler_params=pltpu.CompilerParams(
            dimension_semantics=("parallel",)),
    )(xg, sqc, sqr, m3, *consts)
    return y.reshape(N, hid), pool.reshape(B, hid)


def _head_body(h_ref, s_ref, w1_ref, b1_ref, g1_ref, bt1_ref,
               w2_ref, b2_ref, g2_ref, bt2_ref, w3_ref, b3_ref, out_ref):
    s = s_ref[0, 0]
    h = _cdot(h_ref[...], w1_ref[...]) + b1_ref[...]
    h = jnp.maximum(g1_ref[...] * h / s + bt1_ref[...], 0.0)
    h = _cdot(h, w2_ref[...]) + b2_ref[...]
    h = jnp.maximum(g2_ref[...] * h / s + bt2_ref[...], 0.0)
    out_ref[...] = _cdot(h, w3_ref[...]) + b3_ref[...]


@jax.jit
def kernel(pos, normals, batch, params):
    del batch
    x = jnp.concatenate([pos, normals], axis=1)
    s = jnp.sqrt(1.0 + EPS).astype(jnp.float32).reshape(1, 1)

    pools = []
    for i in (1, 2, 3, 4):
        c1, c2 = params[f'conv{i}']
        fl = params[f'film{i}']
        x, p = _edge_layer(
            x, s, c1['W'], c1['b'][None, :], c1['g'][None, :],
            c1['beta'][None, :], c2['W'], c2['b'][None, :],
            c2['g'][None, :], c2['beta'][None, :],
            fl['Wg'], fl['bg'][None, :], fl['Wb'], fl['bb'][None, :])
        pools.append(p)

    hcat = jnp.concatenate(pools, axis=1)
    return pl.pallas_call(
        _head_body,
        out_shape=jax.ShapeDtypeStruct((B, 1), jnp.float32),
    )(hcat, s,
      params['lin1']['W'], params['lin1']['b'][None, :],
      params['bn1']['g'][None, :], params['bn1']['beta'][None, :],
      params['lin2']['W'], params['lin2']['b'][None, :],
      params['bn2']['g'][None, :], params['bn2']['beta'][None, :],
      params['lin3']['W'], params['lin3']['b'][None, :])
